# Initial kernel scaffold; baseline (speedup 1.0000x reference)
import jax
import jax.numpy as jnp
from jax.experimental import pallas as pl


def kernel(x, tok_emb, pos_emb, ln_g, ln_b, W1, b1, W2, b2):
    raise NotImplementedError("write your pallas kernel here")


# speedup vs baseline: 9.8573x; 9.8573x over previous
import functools

import jax
import jax.numpy as jnp
from jax import lax
from jax.experimental import pallas as pl
from jax.experimental.pallas import tpu as pltpu
from jax.experimental.pallas import tpu_sc as plsc

B, L = 4096, 200
V, D, H = 100000, 128, 64
PAD = 8
LP = L + PAD
CHUNK = 104
CPR = LP // CHUNK
NC, NS = 2, 16
NW = NC * NS
RPW = B // NW
NCHUNK = RPW * CPR
NBUF = 4
GROUPS = NCHUNK // NBUF


def _sc_bag(ids3, tok_emb):
    mesh = plsc.VectorSubcoreMesh(core_axis_name="c", subcore_axis_name="s")

    @functools.partial(
        pl.kernel,
        out_type=jax.ShapeDtypeStruct((B, D), jnp.float32),
        mesh=mesh,
        scratch_types=[
            pltpu.VMEM((NCHUNK, CHUNK), jnp.int32),
            pltpu.VMEM((CHUNK, D), jnp.float32),
            pltpu.VMEM((CHUNK, D), jnp.float32),
            pltpu.VMEM((CHUNK, D), jnp.float32),
            pltpu.VMEM((CHUNK, D), jnp.float32),
            pltpu.VMEM((RPW, D), jnp.float32),
            pltpu.SemaphoreType.DMA,
            pltpu.SemaphoreType.DMA,
            pltpu.SemaphoreType.DMA,
            pltpu.SemaphoreType.DMA,
        ],
    )
    def bag(ids_hbm, tok_hbm, out_hbm, ids_v, g0, g1, g2, g3, out_v,
            s0, s1, s2, s3):
        wid = lax.axis_index("s") * NC + lax.axis_index("c")
        pltpu.sync_copy(ids_hbm.at[wid], ids_v)
        bufs = (g0, g1, g2, g3)
        sems = (s0, s1, s2, s3)
        for k in range(NBUF):
            pltpu.async_copy(tok_hbm.at[ids_v.at[k]], bufs[k], sems[k])

        def accumulate(ref, accs):
            def body(i, a):
                base = 4 * i
                out = []
                for c8 in range(D // 16):
                    v = a[c8]
                    for u in range(4):
                        v = v + ref[base + u, pl.ds(c8 * 16, 16)]
                    out.append(v)
                return tuple(out)
            return lax.fori_loop(0, CHUNK // 4, body, accs)

        def group(g, carry):
            for half in range(2):
                accs = tuple(jnp.zeros((16,), jnp.float32) for _ in range(D // 16))
                row = 2 * g + half
                for k in (2 * half, 2 * half + 1):
                    c = NBUF * g + k
                    pltpu.make_async_copy(
                        tok_hbm.at[ids_v.at[c]], bufs[k], sems[k]).wait()
                    accs = accumulate(bufs[k], accs)

                    @pl.when(c + NBUF < NCHUNK)
                    def _():
                        pltpu.async_copy(
                            tok_hbm.at[ids_v.at[c + NBUF]], bufs[k], sems[k])
                for c8 in range(D // 16):
                    out_v[row, pl.ds(c8 * 16, 16)] = accs[c8]
            return carry

        lax.fori_loop(0, GROUPS, group, 0)
        pltpu.sync_copy(out_v, out_hbm.at[pl.ds(wid * RPW, RPW)])

    return bag(ids3, tok_emb)


def _tc_head(x, sums, pos_emb, tok0, pad_rows, ln_g, ln_b, W1, b1, W2, b2):
    grid = 16
    blk = B // grid

    def body(x_ref, sums_ref, pos_ref, tok0_ref, pad_ref, lng_ref, lnb_ref,
             w1_ref, b1_ref, w2_ref, b2_ref, out_ref):
        xb = x_ref[...]
        valid = (xb != 0).astype(jnp.float32)
        cnt = jnp.sum(valid, axis=1, keepdims=True)
        denom = jnp.maximum(cnt, 1.0)
        nzero = jnp.float32(L) - cnt
        pos_term = jnp.dot(valid, pos_ref[...],
                           preferred_element_type=jnp.float32)
        pad_sum = jnp.sum(pad_ref[...], axis=0, keepdims=True)
        pooled = (sums_ref[...] - pad_sum - nzero * tok0_ref[...]
                  + pos_term) / denom
        mu = jnp.mean(pooled, axis=1, keepdims=True)
        var = jnp.mean((pooled - mu) ** 2, axis=1, keepdims=True)
        hn = (pooled - mu) * lax.rsqrt(var + 1e-5) * lng_ref[...] + lnb_ref[...]
        z = jnp.dot(hn, w1_ref[...], preferred_element_type=jnp.float32)
        z = z + b1_ref[...]
        z = 0.5 * z * (1.0 + lax.erf(z * 0.7071067811865476))
        s = jnp.dot(z, w2_ref[...], preferred_element_type=jnp.float32)
        s = s + b2_ref[...]
        out_ref[...] = s

    return pl.pallas_call(
        body,
        grid=(grid,),
        in_specs=[
            pl.BlockSpec((blk, L), lambda i: (i, 0)),
            pl.BlockSpec((blk, D), lambda i: (i, 0)),
            pl.BlockSpec((L, D), lambda i: (0, 0)),
            pl.BlockSpec((1, D), lambda i: (0, 0)),
            pl.BlockSpec((PAD, D), lambda i: (0, 0)),
            pl.BlockSpec((1, D), lambda i: (0, 0)),
            pl.BlockSpec((1, D), lambda i: (0, 0)),
            pl.BlockSpec((D, H), lambda i: (0, 0)),
            pl.BlockSpec((1, H), lambda i: (0, 0)),
            pl.BlockSpec((H, 1), lambda i: (0, 0)),
            pl.BlockSpec((1, 1), lambda i: (0, 0)),
        ],
        out_specs=pl.BlockSpec((blk, 1), lambda i: (i, 0)),
        out_shape=jax.ShapeDtypeStruct((B, 1), jnp.float32),
    )(x, sums, pos_emb, tok0, pad_rows, ln_g, ln_b, W1, b1, W2, b2)


def _tc_center(raw):
    def body(s_ref, o_ref):
        s = s_ref[...]
        o_ref[...] = s - jnp.mean(s)

    return pl.pallas_call(
        body,
        out_shape=jax.ShapeDtypeStruct(raw.shape, raw.dtype),
    )(raw)


def kernel(x, tok_emb, pos_emb, ln_g, ln_b, W1, b1, W2, b2):
    pad_ids = jnp.arange(1, PAD + 1, dtype=jnp.int32)
    xp = jnp.concatenate([x, jnp.broadcast_to(pad_ids, (B, PAD))], axis=1)
    ids3 = xp.reshape(NW, NCHUNK, CHUNK)
    sums = _sc_bag(ids3, tok_emb)
    raw = _tc_head(
        x, sums, pos_emb,
        tok_emb[0:1], tok_emb[1:PAD + 1],
        ln_g.reshape(1, D), ln_b.reshape(1, D),
        W1, b1.reshape(1, H), W2, b2.reshape(1, 1),
    )
    return _tc_center(raw).reshape(B)
